# baseline (speedup 1.0000x reference)
import functools

import jax
import jax.numpy as jnp
from jax import lax
from jax.experimental import pallas as pl
from jax.experimental.pallas import tpu as pltpu
from jax.experimental.pallas import tpu_sc as plsc

TM = 128


def _route_plan_body(x_ref, wrn_ref, brn_ref, r_ref, p_ref, te_ref,
                     eid_s, within_s, totals_s, excl_s, *, T, E, NT, CH):
    i = pl.program_id(0)
    nchunks = T // CH
    xv = x_ref[...]
    both = jnp.dot(xv, wrn_ref[...], preferred_element_type=jnp.float32)
    both = both + brn_ref[...]
    logits = both[:, :E]
    nl = both[:, E:]
    sp = jnp.maximum(nl, 0.0) + jnp.log(1.0 + jnp.exp(-jnp.abs(nl)))
    noisy = logits + r_ref[...] * sp
    m = jnp.max(noisy, axis=1, keepdims=True)
    ei = lax.broadcasted_iota(jnp.int32, noisy.shape, 1)
    cand = jnp.where(noisy == m, ei, E)
    eid_s[pl.ds(i * CH, CH), :] = jnp.min(cand, axis=1, keepdims=True)

    @pl.when(i == nchunks - 1)
    def _plan():
        NCH = T // TM
        r128 = lax.broadcasted_iota(jnp.int32, (TM, TM), 0)
        c128 = lax.broadcasted_iota(jnp.int32, (TM, TM), 1)
        Lmat = (c128 <= r128).astype(jnp.float32)
        eiota = lax.broadcasted_iota(jnp.int32, (TM, E), 1)

        def body1(c, _):
            eb = eid_s[pl.ds(c * TM, TM), :]
            ohb = (eiota == eb).astype(jnp.float32)
            w = jnp.dot(Lmat, ohb, preferred_element_type=jnp.float32)
            within_s[pl.ds(c * TM, TM), :] = w
            totals_s[pl.ds(c, 1), :] = w[TM - 1:TM, :]
            return 0

        lax.fori_loop(0, NCH, body1, 0)

        rN = lax.broadcasted_iota(jnp.int32, (NCH, NCH), 0)
        cN = lax.broadcasted_iota(jnp.int32, (NCH, NCH), 1)
        Amat = (cN < rN).astype(jnp.float32)
        totals = totals_s[...]
        excl = jnp.dot(Amat, totals, preferred_element_type=jnp.float32)
        excl_s[...] = excl

        counts = excl[NCH - 1:NCH, :] + totals[NCH - 1:NCH, :]
        tiles = (counts.astype(jnp.int32) + TM - 1) // TM
        tiles_f = tiles.astype(jnp.float32)
        rE = lax.broadcasted_iota(jnp.int32, (E, E), 0)
        cE = lax.broadcasted_iota(jnp.int32, (E, E), 1)
        Uexc = (rE < cE).astype(jnp.float32)
        Uinc = (rE <= cE).astype(jnp.float32)
        tile_start = jnp.dot(tiles_f, Uexc,
                             preferred_element_type=jnp.float32)
        cum_incl = jnp.dot(tiles_f, Uinc,
                           preferred_element_type=jnp.float32)
        padded_off = tile_start * TM

        jj = lax.broadcasted_iota(jnp.int32, (NT, E), 0).astype(jnp.float32)
        tcnt = jnp.sum((cum_incl <= jj).astype(jnp.int32), axis=1,
                       keepdims=True)
        te_ref[...] = jnp.minimum(tcnt, E - 1)

        def body2(c, _):
            eb = eid_s[pl.ds(c * TM, TM), :]
            ohb = (eiota == eb).astype(jnp.float32)
            w = within_s[pl.ds(c * TM, TM), :]
            ex = excl_s[pl.ds(c, 1), :]
            pos = w + ex - 1.0 + padded_off
            pv = jnp.sum(ohb * pos, axis=1, keepdims=True)
            p_ref[pl.ds(c * TM, TM), :] = pv.astype(jnp.int32)
            return 0

        lax.fori_loop(0, NCH, body2, 0)


def _route_plan_call(x, Wr, br, Wn, bn, R, NT):
    T, D = x.shape
    E = Wr.shape[1]
    CH = min(1024, T)
    NCH = T // TM
    Wrn = jnp.concatenate([Wr, Wn], axis=1)
    brn = jnp.concatenate([br, bn]).reshape(1, 2 * E)
    return pl.pallas_call(
        functools.partial(_route_plan_body, T=T, E=E, NT=NT, CH=CH),
        grid=(T // CH,),
        in_specs=[
            pl.BlockSpec((CH, D), lambda i: (i, 0)),
            pl.BlockSpec((D, 2 * E), lambda i: (0, 0)),
            pl.BlockSpec((1, 2 * E), lambda i: (0, 0)),
            pl.BlockSpec((CH, E), lambda i: (i, 0)),
        ],
        out_specs=[
            pl.BlockSpec((T, 1), lambda i: (0, 0)),
            pl.BlockSpec((NT, 1), lambda i: (0, 0)),
        ],
        out_shape=[
            jax.ShapeDtypeStruct((T, 1), jnp.int32),
            jax.ShapeDtypeStruct((NT, 1), jnp.int32),
        ],
        scratch_shapes=[
            pltpu.VMEM((T, 1), jnp.int32),
            pltpu.VMEM((T, E), jnp.float32),
            pltpu.VMEM((NCH, E), jnp.float32),
            pltpu.VMEM((NCH, E), jnp.float32),
        ],
        compiler_params=pltpu.CompilerParams(
            dimension_semantics=("arbitrary",)),
    )(x, Wrn, brn, R)


def _ffn_body(te_ref, xs_ref, w1_ref, b1_ref, w2_ref, b2_ref, g_ref, bb_ref,
              ys_ref):
    xv = xs_ref[...]
    h = jnp.dot(xv, w1_ref[0], preferred_element_type=jnp.float32)
    h = jnp.maximum(h + b1_ref[0], 0.0)
    o = jnp.dot(h, w2_ref[0], preferred_element_type=jnp.float32)
    o = o + b2_ref[0] + xv
    mu = jnp.mean(o, axis=1, keepdims=True)
    var = jnp.mean((o - mu) ** 2, axis=1, keepdims=True)
    o = (o - mu) / jnp.sqrt(var + 1e-6) * g_ref[0] + bb_ref[0]
    ys_ref[...] = o


def _ffn_call(te, xs, W1, b1, W2, b2, ln_g, ln_b):
    TP, D = xs.shape
    E, _, H = W1.shape
    NT = TP // TM
    grid_spec = pltpu.PrefetchScalarGridSpec(
        num_scalar_prefetch=1,
        grid=(NT,),
        in_specs=[
            pl.BlockSpec((TM, D), lambda i, te: (i, 0)),
            pl.BlockSpec((1, D, H), lambda i, te: (te[i], 0, 0)),
            pl.BlockSpec((1, 1, H), lambda i, te: (te[i], 0, 0)),
            pl.BlockSpec((1, H, D), lambda i, te: (te[i], 0, 0)),
            pl.BlockSpec((1, 1, D), lambda i, te: (te[i], 0, 0)),
            pl.BlockSpec((1, 1, D), lambda i, te: (te[i], 0, 0)),
            pl.BlockSpec((1, 1, D), lambda i, te: (te[i], 0, 0)),
        ],
        out_specs=pl.BlockSpec((TM, D), lambda i, te: (i, 0)),
    )
    return pl.pallas_call(
        _ffn_body,
        grid_spec=grid_spec,
        out_shape=jax.ShapeDtypeStruct((TP, D), jnp.float32),
        compiler_params=pltpu.CompilerParams(
            dimension_semantics=("arbitrary",)),
    )(te, xs, W1, b1.reshape(E, 1, H), W2, b2.reshape(E, 1, D),
      ln_g.reshape(E, 1, D), ln_b.reshape(E, 1, D))


_BLK = 64


def _sc_scatter_rows(x, p, TP):
    T, D = x.shape
    info = plsc.get_sparse_core_info()
    NW = info.num_cores * info.num_subcores
    per_w = T // NW
    nblk = per_w // _BLK
    mesh = plsc.VectorSubcoreMesh(core_axis_name="c", subcore_axis_name="s")

    @functools.partial(
        pl.kernel, mesh=mesh,
        out_type=jax.ShapeDtypeStruct((TP, D), jnp.float32),
        scratch_types=[
            pltpu.VMEM((_BLK,), jnp.int32),
            pltpu.VMEM((_BLK, D), jnp.float32),
            pltpu.SemaphoreType.DMA,
        ],
    )
    def k(x_hbm, p_hbm, xs_hbm, idx_v, rows_v, sem):
        wid = lax.axis_index("s") * info.num_cores + lax.axis_index("c")
        for b in range(nblk):
            base = wid * per_w + b * _BLK
            pltpu.sync_copy(p_hbm.at[pl.ds(base, _BLK)], idx_v)
            pltpu.sync_copy(x_hbm.at[pl.ds(base, _BLK)], rows_v)
            pltpu.async_copy(rows_v, xs_hbm.at[idx_v], sem).wait()

    return k(x, p)


def _sc_gather_rows(ys, p, T):
    D = ys.shape[1]
    info = plsc.get_sparse_core_info()
    NW = info.num_cores * info.num_subcores
    per_w = T // NW
    nblk = per_w // _BLK
    mesh = plsc.VectorSubcoreMesh(core_axis_name="c", subcore_axis_name="s")

    @functools.partial(
        pl.kernel, mesh=mesh,
        out_type=jax.ShapeDtypeStruct((T, D), jnp.float32),
        scratch_types=[
            pltpu.VMEM((_BLK,), jnp.int32),
            pltpu.VMEM((_BLK, D), jnp.float32),
            pltpu.SemaphoreType.DMA,
        ],
    )
    def k(ys_hbm, p_hbm, out_hbm, idx_v, rows_v, sem):
        wid = lax.axis_index("s") * info.num_cores + lax.axis_index("c")
        for b in range(nblk):
            base = wid * per_w + b * _BLK
            pltpu.sync_copy(p_hbm.at[pl.ds(base, _BLK)], idx_v)
            pltpu.async_copy(ys_hbm.at[idx_v], rows_v, sem).wait()
            pltpu.sync_copy(rows_v, out_hbm.at[pl.ds(base, _BLK)])

    return k(ys, p)


def kernel(x, Wr, br, Wn, bn, W1, b1, W2, b2, ln_g, ln_b):
    T, D = x.shape
    E = Wr.shape[1]
    NT = T // TM + E
    TP = NT * TM

    R = jax.random.normal(jax.random.key(42), (T, E), jnp.float32)
    p2, te2 = _route_plan_call(x, Wr, br, Wn, bn, R, NT)
    p = p2.reshape(T)
    te = te2.reshape(NT)
    xs = _sc_scatter_rows(x, p, TP)
    ys = _ffn_call(te, xs, W1, b1, W2, b2, ln_g, ln_b)
    return _sc_gather_rows(ys, p, T)


# speedup vs baseline: 1.0776x; 1.0776x over previous
import functools

import jax
import jax.numpy as jnp
from jax import lax
from jax.experimental import pallas as pl
from jax.experimental.pallas import tpu as pltpu
from jax.experimental.pallas import tpu_sc as plsc

TM = 128
TMF = 256


def _route_plan_body(x_ref, wrn_ref, brn_ref, r_ref, p_ref, te_ref,
                     eid_s, within_s, totals_s, excl_s, *, T, E, NT, CH):
    i = pl.program_id(0)
    nchunks = T // CH
    xv = x_ref[...]
    both = jnp.dot(xv, wrn_ref[...], preferred_element_type=jnp.float32)
    both = both + brn_ref[...]
    logits = both[:, :E]
    nl = both[:, E:]
    sp = jnp.maximum(nl, 0.0) + jnp.log(1.0 + jnp.exp(-jnp.abs(nl)))
    noisy = logits + r_ref[...] * sp
    m = jnp.max(noisy, axis=1, keepdims=True)
    ei = lax.broadcasted_iota(jnp.int32, noisy.shape, 1)
    cand = jnp.where(noisy == m, ei, E)
    eid_s[pl.ds(i * CH, CH), :] = jnp.min(cand, axis=1, keepdims=True)

    @pl.when(i == nchunks - 1)
    def _plan():
        NCH = T // TM
        r128 = lax.broadcasted_iota(jnp.int32, (TM, TM), 0)
        c128 = lax.broadcasted_iota(jnp.int32, (TM, TM), 1)
        Lmat = (c128 <= r128).astype(jnp.float32)
        eiota = lax.broadcasted_iota(jnp.int32, (TM, E), 1)

        def body1(c, _):
            eb = eid_s[pl.ds(c * TM, TM), :]
            ohb = (eiota == eb).astype(jnp.float32)
            w = jnp.dot(Lmat, ohb, preferred_element_type=jnp.float32)
            within_s[pl.ds(c * TM, TM), :] = w
            totals_s[pl.ds(c, 1), :] = w[TM - 1:TM, :]
            return 0

        lax.fori_loop(0, NCH, body1, 0)

        rN = lax.broadcasted_iota(jnp.int32, (NCH, NCH), 0)
        cN = lax.broadcasted_iota(jnp.int32, (NCH, NCH), 1)
        Amat = (cN < rN).astype(jnp.float32)
        totals = totals_s[...]
        excl = jnp.dot(Amat, totals, preferred_element_type=jnp.float32)
        excl_s[...] = excl

        counts = excl[NCH - 1:NCH, :] + totals[NCH - 1:NCH, :]
        tiles = (counts.astype(jnp.int32) + TMF - 1) // TMF
        tiles_f = tiles.astype(jnp.float32)
        rE = lax.broadcasted_iota(jnp.int32, (E, E), 0)
        cE = lax.broadcasted_iota(jnp.int32, (E, E), 1)
        Uexc = (rE < cE).astype(jnp.float32)
        Uinc = (rE <= cE).astype(jnp.float32)
        tile_start = jnp.dot(tiles_f, Uexc,
                             preferred_element_type=jnp.float32)
        cum_incl = jnp.dot(tiles_f, Uinc,
                           preferred_element_type=jnp.float32)
        padded_off = tile_start * TMF

        jj = lax.broadcasted_iota(jnp.int32, (NT, E), 0).astype(jnp.float32)
        tcnt = jnp.sum((cum_incl <= jj).astype(jnp.int32), axis=1,
                       keepdims=True)
        te_ref[...] = jnp.minimum(tcnt, E - 1)

        def body2(c, _):
            eb = eid_s[pl.ds(c * TM, TM), :]
            ohb = (eiota == eb).astype(jnp.float32)
            w = within_s[pl.ds(c * TM, TM), :]
            ex = excl_s[pl.ds(c, 1), :]
            pos = w + ex - 1.0 + padded_off
            pv = jnp.sum(ohb * pos, axis=1, keepdims=True)
            p_ref[pl.ds(c * TM, TM), :] = pv.astype(jnp.int32)
            return 0

        lax.fori_loop(0, NCH, body2, 0)


def _route_plan_call(x, Wr, br, Wn, bn, R, NT):
    T, D = x.shape
    E = Wr.shape[1]
    CH = min(1024, T)
    NCH = T // TM
    Wrn = jnp.concatenate([Wr, Wn], axis=1)
    brn = jnp.concatenate([br, bn]).reshape(1, 2 * E)
    return pl.pallas_call(
        functools.partial(_route_plan_body, T=T, E=E, NT=NT, CH=CH),
        grid=(T // CH,),
        in_specs=[
            pl.BlockSpec((CH, D), lambda i: (i, 0)),
            pl.BlockSpec((D, 2 * E), lambda i: (0, 0)),
            pl.BlockSpec((1, 2 * E), lambda i: (0, 0)),
            pl.BlockSpec((CH, E), lambda i: (i, 0)),
        ],
        out_specs=[
            pl.BlockSpec((T, 1), lambda i: (0, 0)),
            pl.BlockSpec((NT, 1), lambda i: (0, 0)),
        ],
        out_shape=[
            jax.ShapeDtypeStruct((T, 1), jnp.int32),
            jax.ShapeDtypeStruct((NT, 1), jnp.int32),
        ],
        scratch_shapes=[
            pltpu.VMEM((T, 1), jnp.int32),
            pltpu.VMEM((T, E), jnp.float32),
            pltpu.VMEM((NCH, E), jnp.float32),
            pltpu.VMEM((NCH, E), jnp.float32),
        ],
        compiler_params=pltpu.CompilerParams(
            dimension_semantics=("arbitrary",)),
    )(x, Wrn, brn, R)


def _ffn_body(te_ref, xs_ref, w1_ref, b1_ref, w2_ref, b2_ref, g_ref, bb_ref,
              ys_ref):
    xv = xs_ref[...]
    h = jnp.dot(xv.astype(jnp.bfloat16), w1_ref[0].astype(jnp.bfloat16),
                preferred_element_type=jnp.float32)
    h = jnp.maximum(h + b1_ref[0], 0.0)
    o = jnp.dot(h.astype(jnp.bfloat16), w2_ref[0].astype(jnp.bfloat16),
                preferred_element_type=jnp.float32)
    o = o + b2_ref[0] + xv
    mu = jnp.mean(o, axis=1, keepdims=True)
    var = jnp.mean((o - mu) ** 2, axis=1, keepdims=True)
    o = (o - mu) / jnp.sqrt(var + 1e-6) * g_ref[0] + bb_ref[0]
    ys_ref[...] = o


def _ffn_call(te, xs, W1, b1, W2, b2, ln_g, ln_b):
    TP, D = xs.shape
    E, _, H = W1.shape
    NT = TP // TMF
    grid_spec = pltpu.PrefetchScalarGridSpec(
        num_scalar_prefetch=1,
        grid=(NT,),
        in_specs=[
            pl.BlockSpec((TMF, D), lambda i, te: (i, 0)),
            pl.BlockSpec((1, D, H), lambda i, te: (te[i], 0, 0)),
            pl.BlockSpec((1, 1, H), lambda i, te: (te[i], 0, 0)),
            pl.BlockSpec((1, H, D), lambda i, te: (te[i], 0, 0)),
            pl.BlockSpec((1, 1, D), lambda i, te: (te[i], 0, 0)),
            pl.BlockSpec((1, 1, D), lambda i, te: (te[i], 0, 0)),
            pl.BlockSpec((1, 1, D), lambda i, te: (te[i], 0, 0)),
        ],
        out_specs=pl.BlockSpec((TMF, D), lambda i, te: (i, 0)),
    )
    return pl.pallas_call(
        _ffn_body,
        grid_spec=grid_spec,
        out_shape=jax.ShapeDtypeStruct((TP, D), jnp.float32),
        compiler_params=pltpu.CompilerParams(
            dimension_semantics=("arbitrary",)),
    )(te, xs, W1, b1.reshape(E, 1, H), W2, b2.reshape(E, 1, D),
      ln_g.reshape(E, 1, D), ln_b.reshape(E, 1, D))


_BLK = 64


def _sc_scatter_rows(x, p, TP):
    T, D = x.shape
    info = plsc.get_sparse_core_info()
    NW = info.num_cores * info.num_subcores
    per_w = T // NW
    nblk = per_w // _BLK
    mesh = plsc.VectorSubcoreMesh(core_axis_name="c", subcore_axis_name="s")

    @functools.partial(
        pl.kernel, mesh=mesh,
        out_type=jax.ShapeDtypeStruct((TP, D), jnp.float32),
        scratch_types=[
            pltpu.VMEM((_BLK,), jnp.int32),
            pltpu.VMEM((_BLK, D), jnp.float32),
            pltpu.SemaphoreType.DMA,
        ],
    )
    def k(x_hbm, p_hbm, xs_hbm, idx_v, rows_v, sem):
        wid = lax.axis_index("s") * info.num_cores + lax.axis_index("c")
        for b in range(nblk):
            base = wid * per_w + b * _BLK
            pltpu.sync_copy(p_hbm.at[pl.ds(base, _BLK)], idx_v)
            pltpu.sync_copy(x_hbm.at[pl.ds(base, _BLK)], rows_v)
            pltpu.async_copy(rows_v, xs_hbm.at[idx_v], sem).wait()

    return k(x, p)


def _sc_gather_rows(ys, p, T):
    D = ys.shape[1]
    info = plsc.get_sparse_core_info()
    NW = info.num_cores * info.num_subcores
    per_w = T // NW
    nblk = per_w // _BLK
    mesh = plsc.VectorSubcoreMesh(core_axis_name="c", subcore_axis_name="s")

    @functools.partial(
        pl.kernel, mesh=mesh,
        out_type=jax.ShapeDtypeStruct((T, D), jnp.float32),
        scratch_types=[
            pltpu.VMEM((_BLK,), jnp.int32),
            pltpu.VMEM((_BLK, D), jnp.float32),
            pltpu.SemaphoreType.DMA,
        ],
    )
    def k(ys_hbm, p_hbm, out_hbm, idx_v, rows_v, sem):
        wid = lax.axis_index("s") * info.num_cores + lax.axis_index("c")
        for b in range(nblk):
            base = wid * per_w + b * _BLK
            pltpu.sync_copy(p_hbm.at[pl.ds(base, _BLK)], idx_v)
            pltpu.async_copy(ys_hbm.at[idx_v], rows_v, sem).wait()
            pltpu.sync_copy(rows_v, out_hbm.at[pl.ds(base, _BLK)])

    return k(ys, p)


def kernel(x, Wr, br, Wn, bn, W1, b1, W2, b2, ln_g, ln_b):
    T, D = x.shape
    E = Wr.shape[1]
    NT = T // TMF + E
    TP = NT * TMF

    R = jax.random.normal(jax.random.key(42), (T, E), jnp.float32)
    p2, te2 = _route_plan_call(x, Wr, br, Wn, bn, R, NT)
    p = p2.reshape(T)
    te = te2.reshape(NT)
    xs = _sc_scatter_rows(x, p, TP)
    ys = _ffn_call(te, xs, W1, b1, W2, b2, ln_g, ln_b)
    return _sc_gather_rows(ys, p, T)


# speedup vs baseline: 1.2602x; 1.1695x over previous
import functools

import jax
import jax.numpy as jnp
from jax import lax
from jax.experimental import pallas as pl
from jax.experimental.pallas import tpu as pltpu
from jax.experimental.pallas import tpu_sc as plsc

TM = 128
TMF = 256


def _route_plan_body(x_ref, wrn_ref, brn_ref, r_ref, p_ref, te_ref, vi_ref,
                     eid_s, within_s, totals_s, excl_s, *, T, E, NT, CH):
    i = pl.program_id(0)
    nchunks = T // CH
    xv = x_ref[...]
    both = jnp.dot(xv, wrn_ref[...], preferred_element_type=jnp.float32)
    both = both + brn_ref[...]
    logits = both[:, :E]
    nl = both[:, E:]
    sp = jnp.maximum(nl, 0.0) + jnp.log(1.0 + jnp.exp(-jnp.abs(nl)))
    noisy = logits + r_ref[...] * sp
    m = jnp.max(noisy, axis=1, keepdims=True)
    ei = lax.broadcasted_iota(jnp.int32, noisy.shape, 1)
    cand = jnp.where(noisy == m, ei, E)
    eid_s[pl.ds(i * CH, CH), :] = jnp.min(cand, axis=1, keepdims=True)

    @pl.when(i == nchunks - 1)
    def _plan():
        NCH = T // TM
        r128 = lax.broadcasted_iota(jnp.int32, (TM, TM), 0)
        c128 = lax.broadcasted_iota(jnp.int32, (TM, TM), 1)
        Lmat = (c128 <= r128).astype(jnp.float32)
        eiota = lax.broadcasted_iota(jnp.int32, (TM, E), 1)

        def body1(c, _):
            eb = eid_s[pl.ds(c * TM, TM), :]
            ohb = (eiota == eb).astype(jnp.float32)
            w = jnp.dot(Lmat, ohb, preferred_element_type=jnp.float32)
            within_s[pl.ds(c * TM, TM), :] = w
            totals_s[pl.ds(c, 1), :] = w[TM - 1:TM, :]
            return 0

        lax.fori_loop(0, NCH, body1, 0)

        rN = lax.broadcasted_iota(jnp.int32, (NCH, NCH), 0)
        cN = lax.broadcasted_iota(jnp.int32, (NCH, NCH), 1)
        Amat = (cN < rN).astype(jnp.float32)
        totals = totals_s[...]
        excl = jnp.dot(Amat, totals, preferred_element_type=jnp.float32)
        excl_s[...] = excl

        counts = excl[NCH - 1:NCH, :] + totals[NCH - 1:NCH, :]
        tiles = (counts.astype(jnp.int32) + TMF - 1) // TMF
        tiles_f = tiles.astype(jnp.float32)
        rE = lax.broadcasted_iota(jnp.int32, (E, E), 0)
        cE = lax.broadcasted_iota(jnp.int32, (E, E), 1)
        Uexc = (rE < cE).astype(jnp.float32)
        Uinc = (rE <= cE).astype(jnp.float32)
        tile_start = jnp.dot(tiles_f, Uexc,
                             preferred_element_type=jnp.float32)
        cum_incl = jnp.dot(tiles_f, Uinc,
                           preferred_element_type=jnp.float32)
        padded_off = tile_start * TMF

        jj = lax.broadcasted_iota(jnp.int32, (NT, E), 0).astype(jnp.float32)
        tcnt = jnp.sum((cum_incl <= jj).astype(jnp.int32), axis=1,
                       keepdims=True)
        used = cum_incl[:, E - 1:E].astype(jnp.int32)
        eiota1 = lax.broadcasted_iota(jnp.int32, (1, E), 1)
        laste = jnp.max(jnp.where(counts > 0.0, eiota1, 0), axis=1,
                        keepdims=True)
        jcol = lax.broadcasted_iota(jnp.int32, (NT, 1), 0)
        te_ref[...] = jnp.where(jcol < used,
                                jnp.minimum(tcnt, E - 1), laste)
        vi_ref[...] = jnp.minimum(jcol, used - 1)

        def body2(c, _):
            eb = eid_s[pl.ds(c * TM, TM), :]
            ohb = (eiota == eb).astype(jnp.float32)
            w = within_s[pl.ds(c * TM, TM), :]
            ex = excl_s[pl.ds(c, 1), :]
            pos = w + ex - 1.0 + padded_off
            pv = jnp.sum(ohb * pos, axis=1, keepdims=True)
            p_ref[pl.ds(c * TM, TM), :] = pv.astype(jnp.int32)
            return 0

        lax.fori_loop(0, NCH, body2, 0)


def _route_plan_call(x, Wr, br, Wn, bn, R, NT):
    T, D = x.shape
    E = Wr.shape[1]
    CH = min(1024, T)
    NCH = T // TM
    Wrn = jnp.concatenate([Wr, Wn], axis=1)
    brn = jnp.concatenate([br, bn]).reshape(1, 2 * E)
    return pl.pallas_call(
        functools.partial(_route_plan_body, T=T, E=E, NT=NT, CH=CH),
        grid=(T // CH,),
        in_specs=[
            pl.BlockSpec((CH, D), lambda i: (i, 0)),
            pl.BlockSpec((D, 2 * E), lambda i: (0, 0)),
            pl.BlockSpec((1, 2 * E), lambda i: (0, 0)),
            pl.BlockSpec((CH, E), lambda i: (i, 0)),
        ],
        out_specs=[
            pl.BlockSpec((T, 1), lambda i: (0, 0)),
            pl.BlockSpec((NT, 1), lambda i: (0, 0)),
            pl.BlockSpec((NT, 1), lambda i: (0, 0)),
        ],
        out_shape=[
            jax.ShapeDtypeStruct((T, 1), jnp.int32),
            jax.ShapeDtypeStruct((NT, 1), jnp.int32),
            jax.ShapeDtypeStruct((NT, 1), jnp.int32),
        ],
        scratch_shapes=[
            pltpu.VMEM((T, 1), jnp.int32),
            pltpu.VMEM((T, E), jnp.float32),
            pltpu.VMEM((NCH, E), jnp.float32),
            pltpu.VMEM((NCH, E), jnp.float32),
        ],
        compiler_params=pltpu.CompilerParams(
            dimension_semantics=("arbitrary",)),
    )(x, Wrn, brn, R)


def _ffn_body(te_ref, vi_ref, xs_ref, w1_ref, b1_ref, w2_ref, b2_ref, g_ref,
              bb_ref, ys_ref):
    i = pl.program_id(0)

    @pl.when(vi_ref[i] == i)
    def _compute():
        xv = xs_ref[...]
        h = jnp.dot(xv.astype(jnp.bfloat16), w1_ref[0].astype(jnp.bfloat16),
                    preferred_element_type=jnp.float32)
        h = jnp.maximum(h + b1_ref[0], 0.0)
        o = jnp.dot(h.astype(jnp.bfloat16), w2_ref[0].astype(jnp.bfloat16),
                    preferred_element_type=jnp.float32)
        o = o + b2_ref[0] + xv
        mu = jnp.mean(o, axis=1, keepdims=True)
        var = jnp.mean((o - mu) ** 2, axis=1, keepdims=True)
        o = (o - mu) / jnp.sqrt(var + 1e-6) * g_ref[0] + bb_ref[0]
        ys_ref[...] = o


def _ffn_call(te, vi, xs, W1, b1, W2, b2, ln_g, ln_b):
    TP, D = xs.shape
    E, _, H = W1.shape
    NT = TP // TMF
    grid_spec = pltpu.PrefetchScalarGridSpec(
        num_scalar_prefetch=2,
        grid=(NT,),
        in_specs=[
            pl.BlockSpec((TMF, D), lambda i, te, vi: (vi[i], 0)),
            pl.BlockSpec((1, D, H), lambda i, te, vi: (te[i], 0, 0)),
            pl.BlockSpec((1, 1, H), lambda i, te, vi: (te[i], 0, 0)),
            pl.BlockSpec((1, H, D), lambda i, te, vi: (te[i], 0, 0)),
            pl.BlockSpec((1, 1, D), lambda i, te, vi: (te[i], 0, 0)),
            pl.BlockSpec((1, 1, D), lambda i, te, vi: (te[i], 0, 0)),
            pl.BlockSpec((1, 1, D), lambda i, te, vi: (te[i], 0, 0)),
        ],
        out_specs=pl.BlockSpec((TMF, D), lambda i, te, vi: (vi[i], 0)),
    )
    return pl.pallas_call(
        _ffn_body,
        grid_spec=grid_spec,
        out_shape=jax.ShapeDtypeStruct((TP, D), jnp.float32),
        compiler_params=pltpu.CompilerParams(
            dimension_semantics=("arbitrary",)),
    )(te, vi, xs, W1, b1.reshape(E, 1, H), W2, b2.reshape(E, 1, D),
      ln_g.reshape(E, 1, D), ln_b.reshape(E, 1, D))


_BLK = 64


def _sc_scatter_rows(x, p, TP):
    T, D = x.shape
    info = plsc.get_sparse_core_info()
    NW = info.num_cores * info.num_subcores
    per_w = T // NW
    nblk = per_w // _BLK
    mesh = plsc.VectorSubcoreMesh(core_axis_name="c", subcore_axis_name="s")

    @functools.partial(
        pl.kernel, mesh=mesh,
        out_type=jax.ShapeDtypeStruct((TP, D), jnp.float32),
        scratch_types=[
            pltpu.VMEM((_BLK,), jnp.int32),
            pltpu.VMEM((_BLK, D), jnp.float32),
            pltpu.SemaphoreType.DMA,
        ],
    )
    def k(x_hbm, p_hbm, xs_hbm, idx_v, rows_v, sem):
        wid = lax.axis_index("s") * info.num_cores + lax.axis_index("c")
        for b in range(nblk):
            base = wid * per_w + b * _BLK
            pltpu.sync_copy(p_hbm.at[pl.ds(base, _BLK)], idx_v)
            pltpu.sync_copy(x_hbm.at[pl.ds(base, _BLK)], rows_v)
            pltpu.async_copy(rows_v, xs_hbm.at[idx_v], sem).wait()

    return k(x, p)


def _sc_gather_rows(ys, p, T):
    D = ys.shape[1]
    info = plsc.get_sparse_core_info()
    NW = info.num_cores * info.num_subcores
    per_w = T // NW
    nblk = per_w // _BLK
    mesh = plsc.VectorSubcoreMesh(core_axis_name="c", subcore_axis_name="s")

    @functools.partial(
        pl.kernel, mesh=mesh,
        out_type=jax.ShapeDtypeStruct((T, D), jnp.float32),
        scratch_types=[
            pltpu.VMEM((_BLK,), jnp.int32),
            pltpu.VMEM((_BLK, D), jnp.float32),
            pltpu.SemaphoreType.DMA,
        ],
    )
    def k(ys_hbm, p_hbm, out_hbm, idx_v, rows_v, sem):
        wid = lax.axis_index("s") * info.num_cores + lax.axis_index("c")
        for b in range(nblk):
            base = wid * per_w + b * _BLK
            pltpu.sync_copy(p_hbm.at[pl.ds(base, _BLK)], idx_v)
            pltpu.async_copy(ys_hbm.at[idx_v], rows_v, sem).wait()
            pltpu.sync_copy(rows_v, out_hbm.at[pl.ds(base, _BLK)])

    return k(ys, p)


def kernel(x, Wr, br, Wn, bn, W1, b1, W2, b2, ln_g, ln_b):
    T, D = x.shape
    E = Wr.shape[1]
    NT = T // TMF + E
    TP = NT * TMF

    R = jax.random.normal(jax.random.key(42), (T, E), jnp.float32)
    p2, te2, vi2 = _route_plan_call(x, Wr, br, Wn, bn, R, NT)
    p = p2.reshape(T)
    te = te2.reshape(NT)
    vi = vi2.reshape(NT)
    xs = _sc_scatter_rows(x, p, TP)
    ys = _ffn_call(te, vi, xs, W1, b1, W2, b2, ln_g, ln_b)
    return _sc_gather_rows(ys, p, T)


# speedup vs baseline: 1.2940x; 1.0269x over previous
import functools

import jax
import jax.numpy as jnp
from jax import lax
from jax.experimental import pallas as pl
from jax.experimental.pallas import tpu as pltpu
from jax.experimental.pallas import tpu_sc as plsc

TM = 512
TMF = 256


def _route_plan_body(x_ref, wrn_ref, brn_ref, r_ref, p_ref, te_ref, vi_ref,
                     eid_s, within_s, totals_s, excl_s, *, T, E, NT, CH):
    i = pl.program_id(0)
    nchunks = T // CH
    xv = x_ref[...]
    both = jnp.dot(xv, wrn_ref[...], preferred_element_type=jnp.float32)
    both = both + brn_ref[...]
    logits = both[:, :E]
    nl = both[:, E:]
    sp = jnp.maximum(nl, 0.0) + jnp.log(1.0 + jnp.exp(-jnp.abs(nl)))
    noisy = logits + r_ref[...] * sp
    m = jnp.max(noisy, axis=1, keepdims=True)
    ei = lax.broadcasted_iota(jnp.int32, noisy.shape, 1)
    cand = jnp.where(noisy == m, ei, E)
    eid_s[pl.ds(i * CH, CH), :] = jnp.min(cand, axis=1, keepdims=True)

    @pl.when(i == nchunks - 1)
    def _plan():
        NCH = T // TM
        r128 = lax.broadcasted_iota(jnp.int32, (TM, TM), 0)
        c128 = lax.broadcasted_iota(jnp.int32, (TM, TM), 1)
        Lmat = (c128 <= r128).astype(jnp.float32)
        eiota = lax.broadcasted_iota(jnp.int32, (TM, E), 1)

        def body1(c, _):
            eb = eid_s[pl.ds(c * TM, TM), :]
            ohb = (eiota == eb).astype(jnp.float32)
            w = jnp.dot(Lmat, ohb, preferred_element_type=jnp.float32)
            within_s[pl.ds(c * TM, TM), :] = w
            totals_s[pl.ds(c, 1), :] = w[TM - 1:TM, :]
            return 0

        lax.fori_loop(0, NCH, body1, 0)

        rN = lax.broadcasted_iota(jnp.int32, (NCH, NCH), 0)
        cN = lax.broadcasted_iota(jnp.int32, (NCH, NCH), 1)
        Amat = (cN < rN).astype(jnp.float32)
        totals = totals_s[...]
        excl = jnp.dot(Amat, totals, preferred_element_type=jnp.float32)
        excl_s[...] = excl

        counts = excl[NCH - 1:NCH, :] + totals[NCH - 1:NCH, :]
        tiles = (counts.astype(jnp.int32) + TMF - 1) // TMF
        tiles_f = tiles.astype(jnp.float32)
        rE = lax.broadcasted_iota(jnp.int32, (E, E), 0)
        cE = lax.broadcasted_iota(jnp.int32, (E, E), 1)
        Uexc = (rE < cE).astype(jnp.float32)
        Uinc = (rE <= cE).astype(jnp.float32)
        tile_start = jnp.dot(tiles_f, Uexc,
                             preferred_element_type=jnp.float32)
        cum_incl = jnp.dot(tiles_f, Uinc,
                           preferred_element_type=jnp.float32)
        padded_off = tile_start * TMF

        jj = lax.broadcasted_iota(jnp.int32, (NT, E), 0).astype(jnp.float32)
        tcnt = jnp.sum((cum_incl <= jj).astype(jnp.int32), axis=1,
                       keepdims=True)
        used = cum_incl[:, E - 1:E].astype(jnp.int32)
        eiota1 = lax.broadcasted_iota(jnp.int32, (1, E), 1)
        laste = jnp.max(jnp.where(counts > 0.0, eiota1, 0), axis=1,
                        keepdims=True)
        jcol = lax.broadcasted_iota(jnp.int32, (NT, 1), 0)
        te_ref[...] = jnp.where(jcol < used,
                                jnp.minimum(tcnt, E - 1), laste)
        vi_ref[...] = jnp.minimum(jcol, used - 1)

        def body2(c, _):
            eb = eid_s[pl.ds(c * TM, TM), :]
            ohb = (eiota == eb).astype(jnp.float32)
            w = within_s[pl.ds(c * TM, TM), :]
            ex = excl_s[pl.ds(c, 1), :]
            pos = w + ex - 1.0 + padded_off
            pv = jnp.sum(ohb * pos, axis=1, keepdims=True)
            p_ref[pl.ds(c * TM, TM), :] = pv.astype(jnp.int32)
            return 0

        lax.fori_loop(0, NCH, body2, 0)


def _route_plan_call(x, Wr, br, Wn, bn, R, NT):
    T, D = x.shape
    E = Wr.shape[1]
    CH = min(1024, T)
    NCH = T // TM
    Wrn = jnp.concatenate([Wr, Wn], axis=1)
    brn = jnp.concatenate([br, bn]).reshape(1, 2 * E)
    return pl.pallas_call(
        functools.partial(_route_plan_body, T=T, E=E, NT=NT, CH=CH),
        grid=(T // CH,),
        in_specs=[
            pl.BlockSpec((CH, D), lambda i: (i, 0)),
            pl.BlockSpec((D, 2 * E), lambda i: (0, 0)),
            pl.BlockSpec((1, 2 * E), lambda i: (0, 0)),
            pl.BlockSpec((CH, E), lambda i: (i, 0)),
        ],
        out_specs=[
            pl.BlockSpec((T, 1), lambda i: (0, 0)),
            pl.BlockSpec((NT, 1), lambda i: (0, 0)),
            pl.BlockSpec((NT, 1), lambda i: (0, 0)),
        ],
        out_shape=[
            jax.ShapeDtypeStruct((T, 1), jnp.int32),
            jax.ShapeDtypeStruct((NT, 1), jnp.int32),
            jax.ShapeDtypeStruct((NT, 1), jnp.int32),
        ],
        scratch_shapes=[
            pltpu.VMEM((T, 1), jnp.int32),
            pltpu.VMEM((T, E), jnp.float32),
            pltpu.VMEM((NCH, E), jnp.float32),
            pltpu.VMEM((NCH, E), jnp.float32),
        ],
        compiler_params=pltpu.CompilerParams(
            dimension_semantics=("arbitrary",)),
    )(x, Wrn, brn, R)


def _ffn_body(te_ref, vi_ref, xs_ref, w1_ref, b1_ref, w2_ref, b2_ref, g_ref,
              bb_ref, ys_ref):
    i = pl.program_id(0)

    @pl.when(vi_ref[i] == i)
    def _compute():
        xv = xs_ref[...]
        h = jnp.dot(xv.astype(jnp.bfloat16), w1_ref[0].astype(jnp.bfloat16),
                    preferred_element_type=jnp.float32)
        h = jnp.maximum(h + b1_ref[0], 0.0)
        o = jnp.dot(h.astype(jnp.bfloat16), w2_ref[0].astype(jnp.bfloat16),
                    preferred_element_type=jnp.float32)
        o = o + b2_ref[0] + xv
        mu = jnp.mean(o, axis=1, keepdims=True)
        var = jnp.mean((o - mu) ** 2, axis=1, keepdims=True)
        o = (o - mu) / jnp.sqrt(var + 1e-6) * g_ref[0] + bb_ref[0]
        ys_ref[...] = o


def _ffn_call(te, vi, xs, W1, b1, W2, b2, ln_g, ln_b):
    TP, D = xs.shape
    E, _, H = W1.shape
    NT = TP // TMF
    grid_spec = pltpu.PrefetchScalarGridSpec(
        num_scalar_prefetch=2,
        grid=(NT,),
        in_specs=[
            pl.BlockSpec((TMF, D), lambda i, te, vi: (vi[i], 0)),
            pl.BlockSpec((1, D, H), lambda i, te, vi: (te[i], 0, 0)),
            pl.BlockSpec((1, 1, H), lambda i, te, vi: (te[i], 0, 0)),
            pl.BlockSpec((1, H, D), lambda i, te, vi: (te[i], 0, 0)),
            pl.BlockSpec((1, 1, D), lambda i, te, vi: (te[i], 0, 0)),
            pl.BlockSpec((1, 1, D), lambda i, te, vi: (te[i], 0, 0)),
            pl.BlockSpec((1, 1, D), lambda i, te, vi: (te[i], 0, 0)),
        ],
        out_specs=pl.BlockSpec((TMF, D), lambda i, te, vi: (vi[i], 0)),
    )
    return pl.pallas_call(
        _ffn_body,
        grid_spec=grid_spec,
        out_shape=jax.ShapeDtypeStruct((TP, D), jnp.float32),
        compiler_params=pltpu.CompilerParams(
            dimension_semantics=("arbitrary",)),
    )(te, vi, xs, W1, b1.reshape(E, 1, H), W2, b2.reshape(E, 1, D),
      ln_g.reshape(E, 1, D), ln_b.reshape(E, 1, D))


_BLK = 64


def _sc_scatter_rows(x, p, TP):
    T, D = x.shape
    info = plsc.get_sparse_core_info()
    NW = info.num_cores * info.num_subcores
    per_w = T // NW
    nblk = per_w // _BLK
    mesh = plsc.VectorSubcoreMesh(core_axis_name="c", subcore_axis_name="s")

    @functools.partial(
        pl.kernel, mesh=mesh,
        out_type=jax.ShapeDtypeStruct((TP, D), jnp.float32),
        scratch_types=[
            pltpu.VMEM((_BLK,), jnp.int32),
            pltpu.VMEM((_BLK, D), jnp.float32),
            pltpu.SemaphoreType.DMA,
        ],
    )
    def k(x_hbm, p_hbm, xs_hbm, idx_v, rows_v, sem):
        wid = lax.axis_index("s") * info.num_cores + lax.axis_index("c")
        for b in range(nblk):
            base = wid * per_w + b * _BLK
            pltpu.sync_copy(p_hbm.at[pl.ds(base, _BLK)], idx_v)
            pltpu.sync_copy(x_hbm.at[pl.ds(base, _BLK)], rows_v)
            pltpu.async_copy(rows_v, xs_hbm.at[idx_v], sem).wait()

    return k(x, p)


def _sc_gather_rows(ys, p, T):
    D = ys.shape[1]
    info = plsc.get_sparse_core_info()
    NW = info.num_cores * info.num_subcores
    per_w = T // NW
    nblk = per_w // _BLK
    mesh = plsc.VectorSubcoreMesh(core_axis_name="c", subcore_axis_name="s")

    @functools.partial(
        pl.kernel, mesh=mesh,
        out_type=jax.ShapeDtypeStruct((T, D), jnp.float32),
        scratch_types=[
            pltpu.VMEM((_BLK,), jnp.int32),
            pltpu.VMEM((_BLK, D), jnp.float32),
            pltpu.SemaphoreType.DMA,
        ],
    )
    def k(ys_hbm, p_hbm, out_hbm, idx_v, rows_v, sem):
        wid = lax.axis_index("s") * info.num_cores + lax.axis_index("c")
        for b in range(nblk):
            base = wid * per_w + b * _BLK
            pltpu.sync_copy(p_hbm.at[pl.ds(base, _BLK)], idx_v)
            pltpu.async_copy(ys_hbm.at[idx_v], rows_v, sem).wait()
            pltpu.sync_copy(rows_v, out_hbm.at[pl.ds(base, _BLK)])

    return k(ys, p)


def kernel(x, Wr, br, Wn, bn, W1, b1, W2, b2, ln_g, ln_b):
    T, D = x.shape
    E = Wr.shape[1]
    NT = T // TMF + E
    TP = NT * TMF

    R = jax.random.normal(jax.random.key(42), (T, E), jnp.float32)
    p2, te2, vi2 = _route_plan_call(x, Wr, br, Wn, bn, R, NT)
    p = p2.reshape(T)
    te = te2.reshape(NT)
    vi = vi2.reshape(NT)
    xs = _sc_scatter_rows(x, p, TP)
    ys = _ffn_call(te, vi, xs, W1, b1, W2, b2, ln_g, ln_b)
    return _sc_gather_rows(ys, p, T)


# speedup vs baseline: 1.3039x; 1.0077x over previous
import functools

import jax
import jax.numpy as jnp
from jax import lax
from jax.experimental import pallas as pl
from jax.experimental.pallas import tpu as pltpu
from jax.experimental.pallas import tpu_sc as plsc

TM = 512
TMF = 256


def _route_plan_body(x_ref, wrn_ref, brn_ref, r_ref, p_ref, te_ref, vi_ref,
                     eid_s, within_s, totals_s, excl_s, *, T, E, NT, CH):
    i = pl.program_id(0)
    nchunks = T // CH
    xv = x_ref[...]
    both = jnp.dot(xv, wrn_ref[...], preferred_element_type=jnp.float32)
    both = both + brn_ref[...]
    logits = both[:, :E]
    nl = both[:, E:]
    sp = jnp.maximum(nl, 0.0) + jnp.log(1.0 + jnp.exp(-jnp.abs(nl)))
    noisy = logits + r_ref[...] * sp
    m = jnp.max(noisy, axis=1, keepdims=True)
    ei = lax.broadcasted_iota(jnp.int32, noisy.shape, 1)
    cand = jnp.where(noisy == m, ei, E)
    eid_s[pl.ds(i * CH, CH), :] = jnp.min(cand, axis=1, keepdims=True)

    @pl.when(i == nchunks - 1)
    def _plan():
        NCH = T // TM
        r128 = lax.broadcasted_iota(jnp.int32, (TM, TM), 0)
        c128 = lax.broadcasted_iota(jnp.int32, (TM, TM), 1)
        Lmat = (c128 <= r128).astype(jnp.float32)
        eiota = lax.broadcasted_iota(jnp.int32, (TM, E), 1)

        def body1(c, _):
            eb = eid_s[pl.ds(c * TM, TM), :]
            ohb = (eiota == eb).astype(jnp.float32)
            w = jnp.dot(Lmat, ohb, preferred_element_type=jnp.float32)
            within_s[pl.ds(c * TM, TM), :] = w
            totals_s[pl.ds(c, 1), :] = w[TM - 1:TM, :]
            return 0

        lax.fori_loop(0, NCH, body1, 0)

        rN = lax.broadcasted_iota(jnp.int32, (NCH, NCH), 0)
        cN = lax.broadcasted_iota(jnp.int32, (NCH, NCH), 1)
        Amat = (cN < rN).astype(jnp.float32)
        totals = totals_s[...]
        excl = jnp.dot(Amat, totals, preferred_element_type=jnp.float32)
        excl_s[...] = excl

        counts = excl[NCH - 1:NCH, :] + totals[NCH - 1:NCH, :]
        tiles = (counts.astype(jnp.int32) + TMF - 1) // TMF
        tiles_f = tiles.astype(jnp.float32)
        rE = lax.broadcasted_iota(jnp.int32, (E, E), 0)
        cE = lax.broadcasted_iota(jnp.int32, (E, E), 1)
        Uexc = (rE < cE).astype(jnp.float32)
        Uinc = (rE <= cE).astype(jnp.float32)
        tile_start = jnp.dot(tiles_f, Uexc,
                             preferred_element_type=jnp.float32)
        cum_incl = jnp.dot(tiles_f, Uinc,
                           preferred_element_type=jnp.float32)
        padded_off = tile_start * TMF

        jj = lax.broadcasted_iota(jnp.int32, (NT, E), 0).astype(jnp.float32)
        tcnt = jnp.sum((cum_incl <= jj).astype(jnp.int32), axis=1,
                       keepdims=True)
        used = cum_incl[:, E - 1:E].astype(jnp.int32)
        eiota1 = lax.broadcasted_iota(jnp.int32, (1, E), 1)
        laste = jnp.max(jnp.where(counts > 0.0, eiota1, 0), axis=1,
                        keepdims=True)
        jcol = lax.broadcasted_iota(jnp.int32, (NT, 1), 0)
        te_ref[...] = jnp.where(jcol < used,
                                jnp.minimum(tcnt, E - 1), laste)
        vi_ref[...] = jnp.minimum(jcol, used - 1)

        def body2(c, _):
            eb = eid_s[pl.ds(c * TM, TM), :]
            ohb = (eiota == eb).astype(jnp.float32)
            w = within_s[pl.ds(c * TM, TM), :]
            ex = excl_s[pl.ds(c, 1), :]
            pos = w + ex - 1.0 + padded_off
            pv = jnp.sum(ohb * pos, axis=1, keepdims=True)
            p_ref[pl.ds(c * TM, TM), :] = pv.astype(jnp.int32)
            return 0

        lax.fori_loop(0, NCH, body2, 0)


def _route_plan_call(x, Wr, br, Wn, bn, R, NT):
    T, D = x.shape
    E = Wr.shape[1]
    CH = min(1024, T)
    NCH = T // TM
    Wrn = jnp.concatenate([Wr, Wn], axis=1)
    brn = jnp.concatenate([br, bn]).reshape(1, 2 * E)
    return pl.pallas_call(
        functools.partial(_route_plan_body, T=T, E=E, NT=NT, CH=CH),
        grid=(T // CH,),
        in_specs=[
            pl.BlockSpec((CH, D), lambda i: (i, 0)),
            pl.BlockSpec((D, 2 * E), lambda i: (0, 0)),
            pl.BlockSpec((1, 2 * E), lambda i: (0, 0)),
            pl.BlockSpec((CH, E), lambda i: (i, 0)),
        ],
        out_specs=[
            pl.BlockSpec((T, 1), lambda i: (0, 0)),
            pl.BlockSpec((NT, 1), lambda i: (0, 0)),
            pl.BlockSpec((NT, 1), lambda i: (0, 0)),
        ],
        out_shape=[
            jax.ShapeDtypeStruct((T, 1), jnp.int32),
            jax.ShapeDtypeStruct((NT, 1), jnp.int32),
            jax.ShapeDtypeStruct((NT, 1), jnp.int32),
        ],
        scratch_shapes=[
            pltpu.VMEM((T, 1), jnp.int32),
            pltpu.VMEM((T, E), jnp.float32),
            pltpu.VMEM((NCH, E), jnp.float32),
            pltpu.VMEM((NCH, E), jnp.float32),
        ],
        compiler_params=pltpu.CompilerParams(
            dimension_semantics=("arbitrary",)),
    )(x, Wrn, brn, R)


def _ffn_body(te_ref, vi_ref, xs_ref, w1_ref, b1_ref, w2_ref, b2_ref, g_ref,
              bb_ref, ys_ref):
    i = pl.program_id(0)

    @pl.when(vi_ref[i] == i)
    def _compute():
        xv = xs_ref[...]
        h = jnp.dot(xv.astype(jnp.bfloat16), w1_ref[0].astype(jnp.bfloat16),
                    preferred_element_type=jnp.float32)
        h = jnp.maximum(h + b1_ref[0], 0.0)
        o = jnp.dot(h.astype(jnp.bfloat16), w2_ref[0].astype(jnp.bfloat16),
                    preferred_element_type=jnp.float32)
        o = o + b2_ref[0] + xv
        mu = jnp.mean(o, axis=1, keepdims=True)
        var = jnp.mean((o - mu) ** 2, axis=1, keepdims=True)
        o = (o - mu) / jnp.sqrt(var + 1e-6) * g_ref[0] + bb_ref[0]
        ys_ref[...] = o


def _ffn_call(te, vi, xs, W1, b1, W2, b2, ln_g, ln_b):
    TP, D = xs.shape
    E, _, H = W1.shape
    NT = TP // TMF
    grid_spec = pltpu.PrefetchScalarGridSpec(
        num_scalar_prefetch=2,
        grid=(NT,),
        in_specs=[
            pl.BlockSpec((TMF, D), lambda i, te, vi: (vi[i], 0)),
            pl.BlockSpec((1, D, H), lambda i, te, vi: (te[i], 0, 0)),
            pl.BlockSpec((1, 1, H), lambda i, te, vi: (te[i], 0, 0)),
            pl.BlockSpec((1, H, D), lambda i, te, vi: (te[i], 0, 0)),
            pl.BlockSpec((1, 1, D), lambda i, te, vi: (te[i], 0, 0)),
            pl.BlockSpec((1, 1, D), lambda i, te, vi: (te[i], 0, 0)),
            pl.BlockSpec((1, 1, D), lambda i, te, vi: (te[i], 0, 0)),
        ],
        out_specs=pl.BlockSpec((TMF, D), lambda i, te, vi: (vi[i], 0)),
    )
    return pl.pallas_call(
        _ffn_body,
        grid_spec=grid_spec,
        out_shape=jax.ShapeDtypeStruct((TP, D), jnp.float32),
        compiler_params=pltpu.CompilerParams(
            dimension_semantics=("arbitrary",)),
    )(te, vi, xs, W1, b1.reshape(E, 1, H), W2, b2.reshape(E, 1, D),
      ln_g.reshape(E, 1, D), ln_b.reshape(E, 1, D))


_BLK = 32


def _sc_shuffle(data, p, N_OUT, direction):
    D = data.shape[1]
    T = p.shape[0]
    info = plsc.get_sparse_core_info()
    NW = info.num_cores * info.num_subcores
    per_w = T // NW
    nblk = per_w // _BLK
    p2d = p.reshape(T // _BLK, _BLK)
    mesh = plsc.VectorSubcoreMesh(core_axis_name="c", subcore_axis_name="s")

    @functools.partial(
        pl.kernel, mesh=mesh,
        out_type=jax.ShapeDtypeStruct((N_OUT, D), jnp.float32),
        scratch_types=[
            pltpu.VMEM((nblk, _BLK), jnp.int32),
            pltpu.VMEM((_BLK, D), jnp.float32),
            pltpu.VMEM((_BLK, D), jnp.float32),
            pltpu.SemaphoreType.DMA,
            pltpu.SemaphoreType.DMA,
            pltpu.SemaphoreType.DMA,
            pltpu.SemaphoreType.DMA,
        ],
    )
    def k(data_hbm, p_hbm, out_hbm, idx2, r0, r1, l0, l1, s0, s1):
        wid = lax.axis_index("s") * info.num_cores + lax.axis_index("c")
        pltpu.sync_copy(p_hbm.at[pl.ds(wid * nblk, nblk)], idx2)
        base = wid * per_w
        bufs, lsems, ssems = [r0, r1], [l0, l1], [s0, s1]
        loads, puts = [], []
        for b in range(nblk):
            buf = bufs[b % 2]
            lin = pl.ds(base + b * _BLK, _BLK)
            if direction == "scatter":
                loads.append(pltpu.make_async_copy(
                    data_hbm.at[lin], buf, lsems[b % 2]))
                puts.append(pltpu.make_async_copy(
                    buf, out_hbm.at[idx2.at[b]], ssems[b % 2]))
            else:
                loads.append(pltpu.make_async_copy(
                    data_hbm.at[idx2.at[b]], buf, lsems[b % 2]))
                puts.append(pltpu.make_async_copy(
                    buf, out_hbm.at[lin], ssems[b % 2]))
        loads[0].start()
        for b in range(nblk):
            if b + 1 < nblk:
                if b >= 1:
                    puts[b - 1].wait()
                loads[b + 1].start()
            loads[b].wait()
            puts[b].start()
        if nblk >= 2:
            puts[nblk - 2].wait()
        puts[nblk - 1].wait()

    return k(data, p2d)


def _sc_scatter_rows(x, p, TP):
    return _sc_shuffle(x, p, TP, "scatter")


def _sc_gather_rows(ys, p, T):
    return _sc_shuffle(ys, p, T, "gather")


def kernel(x, Wr, br, Wn, bn, W1, b1, W2, b2, ln_g, ln_b):
    T, D = x.shape
    E = Wr.shape[1]
    NT = T // TMF + E
    TP = NT * TMF

    R = jax.random.normal(jax.random.key(42), (T, E), jnp.float32)
    p2, te2, vi2 = _route_plan_call(x, Wr, br, Wn, bn, R, NT)
    p = p2.reshape(T)
    te = te2.reshape(NT)
    vi = vi2.reshape(NT)
    xs = _sc_scatter_rows(x, p, TP)
    ys = _ffn_call(te, vi, xs, W1, b1, W2, b2, ln_g, ln_b)
    return _sc_gather_rows(ys, p, T)
